# baseline (speedup 1.0000x reference)
import jax
import jax.numpy as jnp
from jax.experimental import pallas as pl
from jax.experimental.pallas import tpu as pltpu

B, N, C = 8, 2048, 128
BLK = 512
NBLK = N // BLK
DB = 128
NDB = N // DB
EPS = 1e-5


def _diag_body(adj_ref, d_ref):
    a = adj_ref[0]
    r = jax.lax.broadcasted_iota(jnp.int32, (DB, DB), 0)
    c = jax.lax.broadcasted_iota(jnp.int32, (DB, DB), 1)
    masked = jnp.where(r == c, a, 0.0)
    d_ref[0] = jnp.dot(
        masked, jnp.ones((DB, C), jnp.float32), preferred_element_type=jnp.float32
    )


def _diag(adj):
    return pl.pallas_call(
        _diag_body,
        grid=(B, NDB),
        in_specs=[pl.BlockSpec((1, DB, DB), lambda b, i: (b, i, i))],
        out_specs=pl.BlockSpec((1, DB, C), lambda b, i: (b, i, 0)),
        out_shape=jax.ShapeDtypeStruct((B, N, C), jnp.float32),
    )(adj)


def _layer_body(adj_ref, h_ref, w_ref, bias_ref, d_ref, r_ref, sum_ref, sq_ref, z_ref):
    b = pl.program_id(0)
    i = pl.program_id(1)

    @pl.when(i == 0)
    def _():
        z_ref[...] = (
            jnp.dot(h_ref[0], w_ref[...], preferred_element_type=jnp.float32)
            + bias_ref[...]
        )

    m = jnp.dot(adj_ref[0], z_ref[...], preferred_element_type=jnp.float32)
    zb = z_ref[pl.ds(i * BLK, BLK), :]
    r = jnp.maximum(m + (1.0 - d_ref[0]) * zb, 0.0)
    r_ref[0] = r

    @pl.when((b == 0) & (i == 0))
    def _():
        sum_ref[...] = jnp.zeros_like(sum_ref)
        sq_ref[...] = jnp.zeros_like(sq_ref)

    sum_ref[...] += jnp.sum(r, axis=0, keepdims=True)
    sq_ref[...] += jnp.sum(r * r, axis=0, keepdims=True)


def _layer(adj, h, w_eff, b_eff, d):
    return pl.pallas_call(
        _layer_body,
        grid=(B, NBLK),
        in_specs=[
            pl.BlockSpec((1, BLK, N), lambda b, i: (b, i, 0)),
            pl.BlockSpec((1, N, C), lambda b, i: (b, 0, 0)),
            pl.BlockSpec((C, C), lambda b, i: (0, 0)),
            pl.BlockSpec((1, C), lambda b, i: (0, 0)),
            pl.BlockSpec((1, BLK, C), lambda b, i: (b, i, 0)),
        ],
        out_specs=[
            pl.BlockSpec((1, BLK, C), lambda b, i: (b, i, 0)),
            pl.BlockSpec((1, C), lambda b, i: (0, 0)),
            pl.BlockSpec((1, C), lambda b, i: (0, 0)),
        ],
        out_shape=[
            jax.ShapeDtypeStruct((B, N, C), jnp.float32),
            jax.ShapeDtypeStruct((1, C), jnp.float32),
            jax.ShapeDtypeStruct((1, C), jnp.float32),
        ],
        scratch_shapes=[pltpu.VMEM((N, C), jnp.float32)],
    )(adj, h, w_eff, b_eff.reshape(1, C), d)


def _affine_body(r_ref, s_ref, t_ref, o_ref):
    o_ref[...] = r_ref[...] * s_ref[...] + t_ref[...]


def _final_affine(r, s, t):
    rf = r.reshape(B * N, C)
    out = pl.pallas_call(
        _affine_body,
        grid=(B * N // 2048,),
        in_specs=[
            pl.BlockSpec((2048, C), lambda i: (i, 0)),
            pl.BlockSpec((1, C), lambda i: (0, 0)),
            pl.BlockSpec((1, C), lambda i: (0, 0)),
        ],
        out_specs=pl.BlockSpec((2048, C), lambda i: (i, 0)),
        out_shape=jax.ShapeDtypeStruct((B * N, C), jnp.float32),
    )(rf, s.reshape(1, C), t.reshape(1, C))
    return out.reshape(B, N, C)


def kernel(x, adj, W0, b0, gamma0, beta0, W1, b1, gamma1, beta1, W2, b2, gamma2, beta2):
    Ws = [W0, W1, W2]
    bs = [b0, b1, b2]
    gammas = [gamma0, gamma1, gamma2]
    betas = [beta0, beta1, beta2]

    d = _diag(adj)
    h = x
    s = t = None
    cnt = float(B * N)
    for l in range(3):
        if l == 0:
            w_eff, b_eff = Ws[0], bs[0]
        else:
            w_eff = s[:, None] * Ws[l]
            b_eff = t @ Ws[l] + bs[l]
        h, sm, sq = _layer(adj, h, w_eff, b_eff, d)
        mean = sm[0] / cnt
        var = sq[0] / cnt - mean * mean
        s = gammas[l] * jax.lax.rsqrt(var + EPS)
        t = betas[l] - mean * s
    return _final_affine(h, s, t)


# speedup vs baseline: 1.3032x; 1.3032x over previous
import jax
import jax.numpy as jnp
from jax.experimental import pallas as pl
from jax.experimental.pallas import tpu as pltpu

B, N, C = 8, 2048, 128
BLK = 512
NBLK = N // BLK
EPS = 1e-5


def _stats_init_and_accum(r, b, i, sum_ref, sq_ref):
    @pl.when((b == 0) & (i == 0))
    def _():
        sum_ref[...] = jnp.zeros_like(sum_ref)
        sq_ref[...] = jnp.zeros_like(sq_ref)

    sum_ref[...] += jnp.sum(r, axis=0, keepdims=True)
    sq_ref[...] += jnp.sum(r * r, axis=0, keepdims=True)


def _layer0_body(adj_ref, h_ref, w_ref, bias_ref, r_ref, sum_ref, sq_ref, d_ref, z_ref):
    b = pl.program_id(0)
    i = pl.program_id(1)

    @pl.when(i == 0)
    def _():
        z_ref[...] = (
            jnp.dot(h_ref[0], w_ref[...], preferred_element_type=jnp.float32)
            + bias_ref[...]
        )

    a = adj_ref[0]
    rows = i * BLK + jax.lax.broadcasted_iota(jnp.int32, (BLK, N), 0)
    cols = jax.lax.broadcasted_iota(jnp.int32, (BLK, N), 1)
    masked = jnp.where(rows == cols, a, 0.0)
    dcol = jnp.dot(
        masked, jnp.ones((N, 1), jnp.float32), preferred_element_type=jnp.float32
    )
    d = jnp.broadcast_to(dcol, (BLK, C))
    d_ref[0] = d

    m = jnp.dot(a, z_ref[...], preferred_element_type=jnp.float32)
    zb = z_ref[pl.ds(i * BLK, BLK), :]
    r = jnp.maximum(m + (1.0 - d) * zb, 0.0)
    r_ref[0] = r
    _stats_init_and_accum(r, b, i, sum_ref, sq_ref)


def _layer_body(adj_ref, h_ref, w_ref, bias_ref, d_ref, r_ref, sum_ref, sq_ref, z_ref):
    b = pl.program_id(0)
    i = pl.program_id(1)

    @pl.when(i == 0)
    def _():
        z_ref[...] = (
            jnp.dot(h_ref[0], w_ref[...], preferred_element_type=jnp.float32)
            + bias_ref[...]
        )

    m = jnp.dot(adj_ref[0], z_ref[...], preferred_element_type=jnp.float32)
    zb = z_ref[pl.ds(i * BLK, BLK), :]
    r = jnp.maximum(m + (1.0 - d_ref[0]) * zb, 0.0)
    r_ref[0] = r
    _stats_init_and_accum(r, b, i, sum_ref, sq_ref)


_ADJ_SPEC = pl.BlockSpec((1, BLK, N), lambda b, i: (b, i, 0))
_H_SPEC = pl.BlockSpec((1, N, C), lambda b, i: (b, 0, 0))
_W_SPEC = pl.BlockSpec((C, C), lambda b, i: (0, 0))
_BIAS_SPEC = pl.BlockSpec((1, C), lambda b, i: (0, 0))
_RBLK_SPEC = pl.BlockSpec((1, BLK, C), lambda b, i: (b, i, 0))
_STAT_SPEC = pl.BlockSpec((1, C), lambda b, i: (0, 0))
_STAT_SHAPE = jax.ShapeDtypeStruct((1, C), jnp.float32)
_BNC_SHAPE = jax.ShapeDtypeStruct((B, N, C), jnp.float32)


def _layer0(adj, h, w_eff, b_eff):
    return pl.pallas_call(
        _layer0_body,
        grid=(B, NBLK),
        in_specs=[_ADJ_SPEC, _H_SPEC, _W_SPEC, _BIAS_SPEC],
        out_specs=[_RBLK_SPEC, _STAT_SPEC, _STAT_SPEC, _RBLK_SPEC],
        out_shape=[_BNC_SHAPE, _STAT_SHAPE, _STAT_SHAPE, _BNC_SHAPE],
        scratch_shapes=[pltpu.VMEM((N, C), jnp.float32)],
    )(adj, h, w_eff, b_eff.reshape(1, C))


def _layer(adj, h, w_eff, b_eff, d):
    return pl.pallas_call(
        _layer_body,
        grid=(B, NBLK),
        in_specs=[_ADJ_SPEC, _H_SPEC, _W_SPEC, _BIAS_SPEC, _RBLK_SPEC],
        out_specs=[_RBLK_SPEC, _STAT_SPEC, _STAT_SPEC],
        out_shape=[_BNC_SHAPE, _STAT_SHAPE, _STAT_SHAPE],
        scratch_shapes=[pltpu.VMEM((N, C), jnp.float32)],
    )(adj, h, w_eff, b_eff.reshape(1, C), d)


def _affine_body(r_ref, s_ref, t_ref, o_ref):
    o_ref[...] = r_ref[...] * s_ref[...] + t_ref[...]


def _final_affine(r, s, t):
    rf = r.reshape(B * N, C)
    out = pl.pallas_call(
        _affine_body,
        grid=(B * N // 2048,),
        in_specs=[
            pl.BlockSpec((2048, C), lambda i: (i, 0)),
            pl.BlockSpec((1, C), lambda i: (0, 0)),
            pl.BlockSpec((1, C), lambda i: (0, 0)),
        ],
        out_specs=pl.BlockSpec((2048, C), lambda i: (i, 0)),
        out_shape=jax.ShapeDtypeStruct((B * N, C), jnp.float32),
    )(rf, s.reshape(1, C), t.reshape(1, C))
    return out.reshape(B, N, C)


def kernel(x, adj, W0, b0, gamma0, beta0, W1, b1, gamma1, beta1, W2, b2, gamma2, beta2):
    Ws = [W0, W1, W2]
    bs = [b0, b1, b2]
    gammas = [gamma0, gamma1, gamma2]
    betas = [beta0, beta1, beta2]

    h = x
    s = t = d = None
    cnt = float(B * N)
    for l in range(3):
        if l == 0:
            w_eff, b_eff = Ws[0], bs[0]
            h, sm, sq, d = _layer0(adj, h, w_eff, b_eff)
        else:
            w_eff = s[:, None] * Ws[l]
            b_eff = t @ Ws[l] + bs[l]
            h, sm, sq = _layer(adj, h, w_eff, b_eff, d)
        mean = sm[0] / cnt
        var = sq[0] / cnt - mean * mean
        s = gammas[l] * jax.lax.rsqrt(var + EPS)
        t = betas[l] - mean * s
    return _final_affine(h, s, t)


# speedup vs baseline: 1.6268x; 1.2483x over previous
import jax
import jax.numpy as jnp
from jax.experimental import pallas as pl
from jax.experimental.pallas import tpu as pltpu

B, N, C = 8, 2048, 128
BLK = 1024
NBLK = N // BLK
EPS = 1e-5


def _stats_init_and_accum(r, b, i, sum_ref, sq_ref):
    @pl.when((b == 0) & (i == 0))
    def _():
        sum_ref[...] = jnp.zeros_like(sum_ref)
        sq_ref[...] = jnp.zeros_like(sq_ref)

    sum_ref[...] += jnp.sum(r, axis=0, keepdims=True)
    sq_ref[...] += jnp.sum(r * r, axis=0, keepdims=True)


def _compute_z(h_ref, w_ref, bias_ref, z_ref):
    z = (
        jnp.dot(h_ref[0], w_ref[...], preferred_element_type=jnp.float32)
        + bias_ref[...]
    )
    z_ref[...] = z.astype(jnp.bfloat16)


def _layer0_body(adj_ref, h_ref, w_ref, bias_ref, r_ref, sum_ref, sq_ref, abf_ref, z_ref):
    b = pl.program_id(0)
    i = pl.program_id(1)

    @pl.when(i == 0)
    def _():
        _compute_z(h_ref, w_ref, bias_ref, z_ref)

    rows = i * BLK + jax.lax.broadcasted_iota(jnp.int32, (BLK, N), 0)
    cols = jax.lax.broadcasted_iota(jnp.int32, (BLK, N), 1)
    abf = jnp.where(rows == cols, 1.0, adj_ref[0]).astype(jnp.bfloat16)
    abf_ref[0] = abf

    m = jnp.dot(abf, z_ref[...], preferred_element_type=jnp.float32)
    r = jnp.maximum(m, 0.0)
    r_ref[0] = r
    _stats_init_and_accum(r, b, i, sum_ref, sq_ref)


def _layer_body(adj_ref, h_ref, w_ref, bias_ref, r_ref, sum_ref, sq_ref, z_ref):
    b = pl.program_id(0)
    i = pl.program_id(1)

    @pl.when(i == 0)
    def _():
        _compute_z(h_ref, w_ref, bias_ref, z_ref)

    m = jnp.dot(adj_ref[0], z_ref[...], preferred_element_type=jnp.float32)
    r = jnp.maximum(m, 0.0)
    r_ref[0] = r
    _stats_init_and_accum(r, b, i, sum_ref, sq_ref)


_ADJ_SPEC = pl.BlockSpec((1, BLK, N), lambda b, i: (b, i, 0))
_H_SPEC = pl.BlockSpec((1, N, C), lambda b, i: (b, 0, 0))
_W_SPEC = pl.BlockSpec((C, C), lambda b, i: (0, 0))
_BIAS_SPEC = pl.BlockSpec((1, C), lambda b, i: (0, 0))
_RBLK_SPEC = pl.BlockSpec((1, BLK, C), lambda b, i: (b, i, 0))
_STAT_SPEC = pl.BlockSpec((1, C), lambda b, i: (0, 0))
_STAT_SHAPE = jax.ShapeDtypeStruct((1, C), jnp.float32)
_BNC_SHAPE = jax.ShapeDtypeStruct((B, N, C), jnp.float32)


def _layer0(adj, h, w_eff, b_eff):
    return pl.pallas_call(
        _layer0_body,
        grid=(B, NBLK),
        in_specs=[_ADJ_SPEC, _H_SPEC, _W_SPEC, _BIAS_SPEC],
        out_specs=[_RBLK_SPEC, _STAT_SPEC, _STAT_SPEC, _ADJ_SPEC],
        out_shape=[
            _BNC_SHAPE,
            _STAT_SHAPE,
            _STAT_SHAPE,
            jax.ShapeDtypeStruct((B, N, N), jnp.bfloat16),
        ],
        scratch_shapes=[pltpu.VMEM((N, C), jnp.bfloat16)],
    )(adj, h, w_eff, b_eff.reshape(1, C))


def _layer(adj_bf, h, w_eff, b_eff):
    return pl.pallas_call(
        _layer_body,
        grid=(B, NBLK),
        in_specs=[_ADJ_SPEC, _H_SPEC, _W_SPEC, _BIAS_SPEC],
        out_specs=[_RBLK_SPEC, _STAT_SPEC, _STAT_SPEC],
        out_shape=[_BNC_SHAPE, _STAT_SHAPE, _STAT_SHAPE],
        scratch_shapes=[pltpu.VMEM((N, C), jnp.bfloat16)],
    )(adj_bf, h, w_eff, b_eff.reshape(1, C))


def _affine_body(r_ref, s_ref, t_ref, o_ref):
    o_ref[...] = r_ref[...] * s_ref[...] + t_ref[...]


def _final_affine(r, s, t):
    rf = r.reshape(B * N, C)
    out = pl.pallas_call(
        _affine_body,
        grid=(B * N // 2048,),
        in_specs=[
            pl.BlockSpec((2048, C), lambda i: (i, 0)),
            pl.BlockSpec((1, C), lambda i: (0, 0)),
            pl.BlockSpec((1, C), lambda i: (0, 0)),
        ],
        out_specs=pl.BlockSpec((2048, C), lambda i: (i, 0)),
        out_shape=jax.ShapeDtypeStruct((B * N, C), jnp.float32),
    )(rf, s.reshape(1, C), t.reshape(1, C))
    return out.reshape(B, N, C)


def kernel(x, adj, W0, b0, gamma0, beta0, W1, b1, gamma1, beta1, W2, b2, gamma2, beta2):
    Ws = [W0, W1, W2]
    bs = [b0, b1, b2]
    gammas = [gamma0, gamma1, gamma2]
    betas = [beta0, beta1, beta2]

    h = x
    s = t = adj_bf = None
    cnt = float(B * N)
    for l in range(3):
        if l == 0:
            h, sm, sq, adj_bf = _layer0(adj, h, Ws[0], bs[0])
        else:
            w_eff = s[:, None] * Ws[l]
            b_eff = t @ Ws[l] + bs[l]
            h, sm, sq = _layer(adj_bf, h, w_eff, b_eff)
        mean = sm[0] / cnt
        var = sq[0] / cnt - mean * mean
        s = gammas[l] * jax.lax.rsqrt(var + EPS)
        t = betas[l] - mean * s
    return _final_affine(h, s, t)


# speedup vs baseline: 1.6326x; 1.0036x over previous
import jax
import jax.numpy as jnp
from jax.experimental import pallas as pl
from jax.experimental.pallas import tpu as pltpu

B, N, C = 8, 2048, 128
BLK = 1024
NBLK = N // BLK
EPS = 1e-5


def _stats_init_and_accum(r, b, i, sum_ref, sq_ref):
    ps = jnp.sum(r, axis=0, keepdims=True)
    pq = jnp.sum(r * r, axis=0, keepdims=True)

    @pl.when(i == 0)
    def _():
        sum_ref[0] = ps
        sq_ref[0] = pq

    @pl.when(i > 0)
    def _():
        sum_ref[0] += ps
        sq_ref[0] += pq


def _compute_z(h_ref, w_ref, bias_ref, z_ref):
    z = (
        jnp.dot(h_ref[0], w_ref[...], preferred_element_type=jnp.float32)
        + bias_ref[...]
    )
    z_ref[...] = z.astype(jnp.bfloat16)


def _layer0_body(adj_ref, h_ref, w_ref, bias_ref, r_ref, sum_ref, sq_ref, abf_ref, z_ref):
    b = pl.program_id(0)
    i = pl.program_id(1)

    @pl.when(i == 0)
    def _():
        _compute_z(h_ref, w_ref, bias_ref, z_ref)

    rows = i * BLK + jax.lax.broadcasted_iota(jnp.int32, (BLK, N), 0)
    cols = jax.lax.broadcasted_iota(jnp.int32, (BLK, N), 1)
    abf = jnp.where(rows == cols, 1.0, adj_ref[0]).astype(jnp.bfloat16)
    abf_ref[0] = abf

    m = jnp.dot(abf, z_ref[...], preferred_element_type=jnp.float32)
    r = jnp.maximum(m, 0.0)
    r_ref[0] = r
    _stats_init_and_accum(r, b, i, sum_ref, sq_ref)


def _layer_body(adj_ref, h_ref, w_ref, bias_ref, r_ref, sum_ref, sq_ref, z_ref):
    b = pl.program_id(0)
    i = pl.program_id(1)

    @pl.when(i == 0)
    def _():
        _compute_z(h_ref, w_ref, bias_ref, z_ref)

    m = jnp.dot(adj_ref[0], z_ref[...], preferred_element_type=jnp.float32)
    r = jnp.maximum(m, 0.0)
    r_ref[0] = r
    _stats_init_and_accum(r, b, i, sum_ref, sq_ref)


_ADJ_SPEC = pl.BlockSpec((1, BLK, N), lambda b, i: (b, i, 0))
_H_SPEC = pl.BlockSpec((1, N, C), lambda b, i: (b, 0, 0))
_W_SPEC = pl.BlockSpec((C, C), lambda b, i: (0, 0))
_BIAS_SPEC = pl.BlockSpec((1, C), lambda b, i: (0, 0))
_RBLK_SPEC = pl.BlockSpec((1, BLK, C), lambda b, i: (b, i, 0))
_STAT_SPEC = pl.BlockSpec((1, 1, C), lambda b, i: (b, 0, 0))
_STAT_SHAPE = jax.ShapeDtypeStruct((B, 1, C), jnp.float32)
_BNC_SHAPE = jax.ShapeDtypeStruct((B, N, C), jnp.float32)


def _layer0(adj, h, w_eff, b_eff):
    return pl.pallas_call(
        _layer0_body,
        grid=(B, NBLK),
        in_specs=[_ADJ_SPEC, _H_SPEC, _W_SPEC, _BIAS_SPEC],
        out_specs=[_RBLK_SPEC, _STAT_SPEC, _STAT_SPEC, _ADJ_SPEC],
        out_shape=[
            _BNC_SHAPE,
            _STAT_SHAPE,
            _STAT_SHAPE,
            jax.ShapeDtypeStruct((B, N, N), jnp.bfloat16),
        ],
        scratch_shapes=[pltpu.VMEM((N, C), jnp.bfloat16)],
        compiler_params=pltpu.CompilerParams(
            dimension_semantics=("parallel", "arbitrary")
        ),
    )(adj, h, w_eff, b_eff.reshape(1, C))


def _layer(adj_bf, h, w_eff, b_eff):
    return pl.pallas_call(
        _layer_body,
        grid=(B, NBLK),
        in_specs=[_ADJ_SPEC, _H_SPEC, _W_SPEC, _BIAS_SPEC],
        out_specs=[_RBLK_SPEC, _STAT_SPEC, _STAT_SPEC],
        out_shape=[_BNC_SHAPE, _STAT_SHAPE, _STAT_SHAPE],
        scratch_shapes=[pltpu.VMEM((N, C), jnp.bfloat16)],
        compiler_params=pltpu.CompilerParams(
            dimension_semantics=("parallel", "arbitrary")
        ),
    )(adj_bf, h, w_eff, b_eff.reshape(1, C))


def _affine_body(r_ref, s_ref, t_ref, o_ref):
    o_ref[...] = r_ref[...] * s_ref[...] + t_ref[...]


def _final_affine(r, s, t):
    rf = r.reshape(B * N, C)
    out = pl.pallas_call(
        _affine_body,
        grid=(B * N // 2048,),
        in_specs=[
            pl.BlockSpec((2048, C), lambda i: (i, 0)),
            pl.BlockSpec((1, C), lambda i: (0, 0)),
            pl.BlockSpec((1, C), lambda i: (0, 0)),
        ],
        out_specs=pl.BlockSpec((2048, C), lambda i: (i, 0)),
        out_shape=jax.ShapeDtypeStruct((B * N, C), jnp.float32),
    )(rf, s.reshape(1, C), t.reshape(1, C))
    return out.reshape(B, N, C)


def kernel(x, adj, W0, b0, gamma0, beta0, W1, b1, gamma1, beta1, W2, b2, gamma2, beta2):
    Ws = [W0, W1, W2]
    bs = [b0, b1, b2]
    gammas = [gamma0, gamma1, gamma2]
    betas = [beta0, beta1, beta2]

    h = x
    s = t = adj_bf = None
    cnt = float(B * N)
    for l in range(3):
        if l == 0:
            h, sm, sq, adj_bf = _layer0(adj, h, Ws[0], bs[0])
        else:
            w_eff = s[:, None] * Ws[l]
            b_eff = t @ Ws[l] + bs[l]
            h, sm, sq = _layer(adj_bf, h, w_eff, b_eff)
        mean = jnp.sum(sm, axis=(0, 1)) / cnt
        var = jnp.sum(sq, axis=(0, 1)) / cnt - mean * mean
        s = gammas[l] * jax.lax.rsqrt(var + EPS)
        t = betas[l] - mean * s
    return _final_affine(h, s, t)
